# baseline (speedup 1.0000x reference)
import jax
import jax.numpy as jnp
from jax import lax
from jax.experimental import pallas as pl
from jax.experimental.pallas import tpu as pltpu

_BLOCK = 1000

_NT_DIMS = (((1,), (1,)), ((), ()))


def _lstm_cell_block(x, h, c, wih, whh, b):
    gates = (
        lax.dot_general(x.astype(jnp.bfloat16), wih, _NT_DIMS,
                        preferred_element_type=jnp.float32)
        + lax.dot_general(h.astype(jnp.bfloat16), whh, _NT_DIMS,
                          preferred_element_type=jnp.float32)
        + b
    )
    d = x.shape[1]
    i = 0.5 * jnp.tanh(0.5 * gates[:, 0:d]) + 0.5
    f = 0.5 * jnp.tanh(0.5 * gates[:, d : 2 * d]) + 0.5
    g = jnp.tanh(gates[:, 2 * d : 3 * d])
    o = 0.5 * jnp.tanh(0.5 * gates[:, 3 * d : 4 * d]) + 0.5
    c_new = f * c + i * g
    h_new = o * jnp.tanh(c_new)
    return h_new, c_new


def _both_types_kernel(xl_ref, hl_ref, cl_ref, xc_ref, hc_ref, cc_ref,
                       wihl_ref, whhl_ref, bihl_ref, bhhl_ref,
                       wihc_ref, whhc_ref, bihc_ref, bhhc_ref, out_ref):
    b_lit = bihl_ref[...] + bhhl_ref[...]
    b_cls = bihc_ref[...] + bhhc_ref[...]
    h_lit, c_lit = _lstm_cell_block(
        xl_ref[...], hl_ref[...], cl_ref[...],
        wihl_ref[...].astype(jnp.bfloat16), whhl_ref[...].astype(jnp.bfloat16),
        b_lit)
    h_cls, c_cls = _lstm_cell_block(
        xc_ref[...], hc_ref[...], cc_ref[...],
        wihc_ref[...].astype(jnp.bfloat16), whhc_ref[...].astype(jnp.bfloat16),
        b_cls)
    out_ref[0] = h_lit
    out_ref[1] = c_lit
    out_ref[2] = h_cls
    out_ref[3] = c_cls


def kernel(x_lit, x_cls, edge_index_lit_to_cls, edge_index_cls_to_lit,
           h_lit, c_lit, h_cls, c_cls,
           W_ih_lit, W_hh_lit, b_ih_lit, b_hh_lit,
           W_ih_cls, W_hh_cls, b_ih_cls, b_hh_cls):
    del edge_index_lit_to_cls, edge_index_cls_to_lit
    n, d = x_lit.shape

    nb = n // _BLOCK
    row_spec = pl.BlockSpec((_BLOCK, d), lambda j: (j, 0))
    w_spec = pl.BlockSpec((4 * d, d), lambda j: (0, 0))
    b_spec = pl.BlockSpec((1, 4 * d), lambda j: (0, 0))
    out = pl.pallas_call(
        _both_types_kernel,
        grid=(nb,),
        in_specs=[
            row_spec, row_spec, row_spec,
            row_spec, row_spec, row_spec,
            w_spec, w_spec, b_spec, b_spec,
            w_spec, w_spec, b_spec, b_spec,
        ],
        out_specs=pl.BlockSpec((4, _BLOCK, d), lambda j: (0, j, 0)),
        out_shape=jax.ShapeDtypeStruct((4, n, d), jnp.float32),
        compiler_params=pltpu.CompilerParams(
            dimension_semantics=("arbitrary",),
        ),
    )(x_lit, h_lit, c_lit, x_cls, h_cls, c_cls,
      W_ih_lit, W_hh_lit, b_ih_lit.reshape(1, 4 * d), b_hh_lit.reshape(1, 4 * d),
      W_ih_cls, W_hh_cls, b_ih_cls.reshape(1, 4 * d), b_hh_cls.reshape(1, 4 * d))
    return out


# speedup vs baseline: 1.1135x; 1.1135x over previous
import jax
import jax.numpy as jnp
from jax import lax
from jax.experimental import pallas as pl
from jax.experimental.pallas import tpu as pltpu

_BLOCK = 2000

_NT_DIMS = (((1,), (1,)), ((), ()))


def _lstm_cell_block(x, h, c, wih, whh, b):
    xh = jnp.concatenate(
        [x.astype(jnp.bfloat16), h.astype(jnp.bfloat16)], axis=1)
    w = jnp.concatenate([wih, whh], axis=1)
    gates = lax.dot_general(xh, w, _NT_DIMS,
                            preferred_element_type=jnp.float32) + b
    d = x.shape[1]
    i = 0.5 * jnp.tanh(0.5 * gates[:, 0:d]) + 0.5
    f = 0.5 * jnp.tanh(0.5 * gates[:, d : 2 * d]) + 0.5
    g = jnp.tanh(gates[:, 2 * d : 3 * d])
    o = 0.5 * jnp.tanh(0.5 * gates[:, 3 * d : 4 * d]) + 0.5
    c_new = f * c + i * g
    h_new = o * jnp.tanh(c_new)
    return h_new, c_new


def _both_types_kernel(xl_ref, hl_ref, cl_ref, xc_ref, hc_ref, cc_ref,
                       wihl_ref, whhl_ref, bihl_ref, bhhl_ref,
                       wihc_ref, whhc_ref, bihc_ref, bhhc_ref, out_ref):
    b_lit = bihl_ref[...] + bhhl_ref[...]
    b_cls = bihc_ref[...] + bhhc_ref[...]
    h_lit, c_lit = _lstm_cell_block(
        xl_ref[...], hl_ref[...], cl_ref[...],
        wihl_ref[...].astype(jnp.bfloat16), whhl_ref[...].astype(jnp.bfloat16),
        b_lit)
    h_cls, c_cls = _lstm_cell_block(
        xc_ref[...], hc_ref[...], cc_ref[...],
        wihc_ref[...].astype(jnp.bfloat16), whhc_ref[...].astype(jnp.bfloat16),
        b_cls)
    out_ref[0] = h_lit
    out_ref[1] = c_lit
    out_ref[2] = h_cls
    out_ref[3] = c_cls


def kernel(x_lit, x_cls, edge_index_lit_to_cls, edge_index_cls_to_lit,
           h_lit, c_lit, h_cls, c_cls,
           W_ih_lit, W_hh_lit, b_ih_lit, b_hh_lit,
           W_ih_cls, W_hh_cls, b_ih_cls, b_hh_cls):
    del edge_index_lit_to_cls, edge_index_cls_to_lit
    n, d = x_lit.shape

    nb = n // _BLOCK
    row_spec = pl.BlockSpec((_BLOCK, d), lambda j: (j, 0))
    w_spec = pl.BlockSpec((4 * d, d), lambda j: (0, 0))
    b_spec = pl.BlockSpec((1, 4 * d), lambda j: (0, 0))
    out = pl.pallas_call(
        _both_types_kernel,
        grid=(nb,),
        in_specs=[
            row_spec, row_spec, row_spec,
            row_spec, row_spec, row_spec,
            w_spec, w_spec, b_spec, b_spec,
            w_spec, w_spec, b_spec, b_spec,
        ],
        out_specs=pl.BlockSpec((4, _BLOCK, d), lambda j: (0, j, 0)),
        out_shape=jax.ShapeDtypeStruct((4, n, d), jnp.float32),
        compiler_params=pltpu.CompilerParams(
            dimension_semantics=("arbitrary",),
        ),
    )(x_lit, h_lit, c_lit, x_cls, h_cls, c_cls,
      W_ih_lit, W_hh_lit, b_ih_lit.reshape(1, 4 * d), b_hh_lit.reshape(1, 4 * d),
      W_ih_cls, W_hh_cls, b_ih_cls.reshape(1, 4 * d), b_hh_cls.reshape(1, 4 * d))
    return out
